# baseline (speedup 1.0000x reference)
import jax
import jax.numpy as jnp
from jax import lax
from jax.experimental import pallas as pl
from jax.experimental.pallas import tpu as pltpu
from jax.experimental.pallas import tpu_sc as plsc

D_MODEL = 64
SCALE = 8.0
NC, NS, L = 2, 16, 16
NW = NC * NS
B_ROWS, S_LEN = 4096, 200
B = B_ROWS * S_LEN
BPW = B // NW
CHUNK = 128
NCHUNK = BPW // CHUNK


def _embed_body(x_hbm, table_hbm, out_hbm, idx_v, rows_v, sem):
    wid = lax.axis_index("s") * NC + lax.axis_index("c")
    base = wid * BPW

    def chunk_body(i, carry):
        off = base + i * CHUNK
        pltpu.sync_copy(x_hbm.at[pl.ds(off, CHUNK)], idx_v)
        pltpu.async_copy(table_hbm.at[idx_v], rows_v, sem).wait()

        def row_body(r, c2):
            for c in range(D_MODEL // L):
                sl = pl.ds(c * L, L)
                rows_v[r, sl] = rows_v[r, sl] * SCALE
            return c2

        lax.fori_loop(0, CHUNK, row_body, 0)
        pltpu.sync_copy(rows_v, out_hbm.at[pl.ds(off, CHUNK)])
        return carry

    lax.fori_loop(0, NCHUNK, chunk_body, 0)


@jax.jit
def kernel(x, table):
    xf = x.reshape(B)
    out = pl.kernel(
        _embed_body,
        out_type=jax.ShapeDtypeStruct((B, D_MODEL), jnp.float32),
        mesh=plsc.VectorSubcoreMesh(
            core_axis_name="c", subcore_axis_name="s",
            num_cores=NC, num_subcores=NS,
        ),
        scratch_types=[
            pltpu.VMEM((CHUNK,), jnp.int32),
            pltpu.VMEM((CHUNK, D_MODEL), jnp.float32),
            pltpu.SemaphoreType.DMA,
        ],
        compiler_params=pltpu.CompilerParams(use_tc_tiling_on_sc=False),
    )(xf, table)
    return out.reshape(B_ROWS, S_LEN, D_MODEL)


# speedup vs baseline: 1.2082x; 1.2082x over previous
import jax
import jax.numpy as jnp
from jax import lax
from jax.experimental import pallas as pl
from jax.experimental.pallas import tpu as pltpu
from jax.experimental.pallas import tpu_sc as plsc

D_MODEL = 64
SCALE = 8.0
NC, NS, L = 2, 16, 16
NW = NC * NS
B_ROWS, S_LEN = 4096, 200
B = B_ROWS * S_LEN
BPW = B // NW
CHUNK = 128
K = 4
GROUP = K * CHUNK
G = BPW // GROUP


def _embed_body(x_hbm, table_hbm, out_hbm, idx_v, rows_v, sem0, sem1):
    wid = lax.axis_index("s") * NC + lax.axis_index("c")
    base = wid * BPW
    sems = (sem0, sem1)

    def fire(g, b):
        off = base + g * GROUP
        pltpu.sync_copy(x_hbm.at[pl.ds(off, GROUP)], idx_v.at[b])
        for j in range(K):
            sl = pl.ds(j * CHUNK, CHUNK)
            pltpu.async_copy(
                table_hbm.at[idx_v.at[b, sl]], rows_v.at[b, sl], sems[b])

    def drain(b):
        for j in range(K):
            sl = pl.ds(j * CHUNK, CHUNK)
            pltpu.make_async_copy(
                table_hbm.at[idx_v.at[b, sl]], rows_v.at[b, sl], sems[b]
            ).wait()

    def scale(b):
        def row_body(r, carry):
            for c in range(D_MODEL // L):
                sl = pl.ds(c * L, L)
                rows_v[b, r, sl] = rows_v[b, r, sl] * SCALE
            return carry

        lax.fori_loop(0, GROUP, row_body, 0)

    def scatter(g, b):
        off = base + g * GROUP
        pltpu.sync_copy(rows_v.at[b], out_hbm.at[pl.ds(off, GROUP)])

    fire(0, 0)

    def body(h, carry):
        g0 = 2 * h

        @pl.when(g0 + 1 < G)
        def _():
            fire(g0 + 1, 1)

        drain(0)
        scale(0)
        scatter(g0, 0)

        @pl.when(g0 + 2 < G)
        def _():
            fire(g0 + 2, 0)

        @pl.when(g0 + 1 < G)
        def _():
            drain(1)
            scale(1)
            scatter(g0 + 1, 1)

        return carry

    lax.fori_loop(0, (G + 1) // 2, body, 0)


@jax.jit
def kernel(x, table):
    xf = x.reshape(B)
    out = pl.kernel(
        _embed_body,
        out_type=jax.ShapeDtypeStruct((B, D_MODEL), jnp.float32),
        mesh=plsc.VectorSubcoreMesh(
            core_axis_name="c", subcore_axis_name="s",
            num_cores=NC, num_subcores=NS,
        ),
        scratch_types=[
            pltpu.VMEM((2, GROUP), jnp.int32),
            pltpu.VMEM((2, GROUP, D_MODEL), jnp.float32),
            pltpu.SemaphoreType.DMA,
            pltpu.SemaphoreType.DMA,
        ],
        compiler_params=pltpu.CompilerParams(use_tc_tiling_on_sc=False),
    )(xf, table)
    return out.reshape(B_ROWS, S_LEN, D_MODEL)


# speedup vs baseline: 1.2768x; 1.0568x over previous
import jax
import jax.numpy as jnp
from jax import lax
from jax.experimental import pallas as pl
from jax.experimental.pallas import tpu as pltpu
from jax.experimental.pallas import tpu_sc as plsc

D_MODEL = 64
SCALE = 8.0
NC, NS, L = 2, 16, 16
NW = NC * NS
B_ROWS, S_LEN = 4096, 200
B = B_ROWS * S_LEN
BPW = B // NW
CHUNK = 128
NCHUNK = BPW // CHUNK
NBUF = 8
LOOKAHEAD = 6
OUTER = NCHUNK // NBUF


def _embed_body(x_hbm, table_hbm, out_hbm, idx_v, rows_v, *sems):
    gsems, ssems = sems[:NBUF], sems[NBUF:]
    wid = lax.axis_index("s") * NC + lax.axis_index("c")
    base = wid * BPW
    pltpu.sync_copy(x_hbm.at[pl.ds(base, BPW)], idx_v)

    def gather_pair(c, b):
        return (table_hbm.at[idx_v.at[pl.ds(c * CHUNK, CHUNK)]],
                rows_v.at[b], gsems[b])

    def scatter_pair(c, b):
        return (rows_v.at[b], out_hbm.at[pl.ds(base + c * CHUNK, CHUNK)],
                ssems[b])

    def scale(b):
        def row_body(r, carry):
            for s in range(D_MODEL // L):
                sl = pl.ds(s * L, L)
                rows_v[b, r, sl] = rows_v[b, r, sl] * SCALE
            return carry

        lax.fori_loop(0, CHUNK, row_body, 0)

    def step(c, b, do_wait_ssem, do_fire):
        pltpu.make_async_copy(*gather_pair(c, b)).wait()
        scale(b)
        pltpu.async_copy(*scatter_pair(c, b))
        f = c + LOOKAHEAD
        bf = (b + LOOKAHEAD) % NBUF
        if do_wait_ssem:
            pltpu.make_async_copy(*scatter_pair(f - NBUF, bf)).wait()
        if do_fire:
            pltpu.async_copy(*gather_pair(f, bf))

    for c in range(LOOKAHEAD):
        pltpu.async_copy(*gather_pair(c, c))

    for b in range(NBUF):
        step(b, b, do_wait_ssem=(b + LOOKAHEAD >= NBUF), do_fire=True)

    def outer(h, carry):
        c0 = h * NBUF
        for b in range(NBUF):
            step(c0 + b, b, do_wait_ssem=True, do_fire=True)
        return carry

    lax.fori_loop(1, OUTER - 1, outer, 0)

    for b in range(NBUF):
        c = NCHUNK - NBUF + b
        step(c, b, do_wait_ssem=True, do_fire=(c + LOOKAHEAD < NCHUNK))

    for c in range(NCHUNK - (NBUF - LOOKAHEAD), NCHUNK):
        pltpu.make_async_copy(*scatter_pair(c, c % NBUF)).wait()


@jax.jit
def kernel(x, table):
    xf = x.reshape(B)
    out = pl.kernel(
        _embed_body,
        out_type=jax.ShapeDtypeStruct((B, D_MODEL), jnp.float32),
        mesh=plsc.VectorSubcoreMesh(
            core_axis_name="c", subcore_axis_name="s",
            num_cores=NC, num_subcores=NS,
        ),
        scratch_types=(
            [pltpu.VMEM((BPW,), jnp.int32),
             pltpu.VMEM((NBUF, CHUNK, D_MODEL), jnp.float32)]
            + [pltpu.SemaphoreType.DMA] * (2 * NBUF)
        ),
        compiler_params=pltpu.CompilerParams(use_tc_tiling_on_sc=False),
    )(xf, table)
    return out.reshape(B_ROWS, S_LEN, D_MODEL)
